# Initial kernel scaffold; baseline (speedup 1.0000x reference)
import jax
import jax.numpy as jnp
from jax.experimental import pallas as pl


def kernel(x, edge_index, batch, W1, b1, W2, b2, We, be, Wm, bm):
    raise NotImplementedError("write your pallas kernel here")


# speedup vs baseline: 14.2725x; 14.2725x over previous
import functools

import jax
import jax.numpy as jnp
from jax import lax
from jax.experimental import pallas as pl
from jax.experimental.pallas import tpu as pltpu
from jax.experimental.pallas import tpu_sc as plsc

N = 10000
E = 320000
D = 128
NOPS = 5

NPAD = 10240
NSUB = 16
NW = 32
CHUNK = 128
NCH = 80
EPW = NCH * CHUNK
EPAD = NW * EPW
EROWS = EPAD // CHUNK
RPS = NPAD // NSUB
BR = 1024
GRID = NPAD // BR

_mesh = plsc.VectorSubcoreMesh(core_axis_name="c", subcore_axis_name="s")
_sc_params = pltpu.CompilerParams(needs_layout_passes=False)


def _zero16():
    return jnp.zeros((16,), jnp.float32)


def _sc_deg_body(dstp, out, ones_v, dbuf, zbuf, acc):
    c = lax.axis_index("c")
    s = lax.axis_index("s")
    w = c * NSUB + s
    for k in range(CHUNK // 16):
        ones_v[pl.ds(k * 16, 16)] = jnp.ones((16,), jnp.float32)
    for k in range(RPS // 16):
        zbuf[pl.ds(k * 16, 16)] = _zero16()
    pltpu.sync_copy(zbuf, acc.at[pl.ds(s * RPS, RPS)])
    plsc.subcore_barrier()
    pltpu.sync_copy(dstp.at[pl.ds(w * NCH, NCH)], dbuf)

    def lp(j, cr):
        pltpu.sync_copy(ones_v, acc.at[dbuf.at[j]], add=True)
        return cr

    lax.fori_loop(0, NCH, lp, 0)
    plsc.subcore_barrier()
    pltpu.sync_copy(acc.at[pl.ds(s * RPS, RPS)], zbuf)
    pltpu.sync_copy(zbuf, out.at[pl.ds(c * NPAD + s * RPS, RPS)])


_sc_deg = pl.kernel(
    _sc_deg_body,
    mesh=_mesh,
    compiler_params=_sc_params,
    out_type=jax.ShapeDtypeStruct((2 * NPAD,), jnp.float32),
    scratch_types=[
        pltpu.VMEM((CHUNK,), jnp.float32),
        pltpu.VMEM((NCH, CHUNK), jnp.int32),
        pltpu.VMEM((RPS,), jnp.float32),
        pltpu.VMEM_SHARED((NPAD,), jnp.float32),
    ],
)


SEGS = 5
SCH = NCH // SEGS


def _sc_mp_body(table, srcp, dstp, out, sbuf, dbuf, rows0, rows1, zbuf, acc,
                sem0, sem1):
    c = lax.axis_index("c")
    s = lax.axis_index("s")
    w = c * NSUB + s
    for i in range(16):
        for k in range(8):
            zbuf[i, pl.ds(k * 16, 16)] = _zero16()

    def zb(k, cr):
        pltpu.sync_copy(zbuf, acc.at[pl.ds(s * RPS + k * 16, 16)])
        return cr

    lax.fori_loop(0, RPS // 16, zb, 0)
    plsc.subcore_barrier()

    def seg(g, cr):
        base = w * NCH + g * SCH
        pltpu.sync_copy(srcp.at[pl.ds(base, SCH)], sbuf)
        pltpu.sync_copy(dstp.at[pl.ds(base, SCH)], dbuf)
        pltpu.async_copy(table.at[sbuf.at[0]], rows0, sem0)

        def lp(t, cr2):
            j0 = 2 * t
            j1 = 2 * t + 1
            pltpu.make_async_copy(table.at[sbuf.at[j0]], rows0, sem0).wait()
            pltpu.async_copy(table.at[sbuf.at[j1]], rows1, sem1)
            pltpu.sync_copy(rows0, acc.at[dbuf.at[j0]], add=True)
            pltpu.make_async_copy(table.at[sbuf.at[j1]], rows1, sem1).wait()

            @pl.when(t + 1 < SCH // 2)
            def _():
                pltpu.async_copy(table.at[sbuf.at[j0 + 2]], rows0, sem0)

            pltpu.sync_copy(rows1, acc.at[dbuf.at[j1]], add=True)
            return cr2

        lax.fori_loop(0, SCH // 2, lp, 0)
        return cr

    lax.fori_loop(0, SEGS, seg, 0)
    plsc.subcore_barrier()

    def wb(k, cr):
        pltpu.sync_copy(acc.at[pl.ds(s * RPS + k * CHUNK, CHUNK)], rows0)
        pltpu.sync_copy(
            rows0, out.at[pl.ds(c * NPAD + s * RPS + k * CHUNK, CHUNK)])
        return cr

    lax.fori_loop(0, RPS // CHUNK, wb, 0)


_sc_mp = pl.kernel(
    _sc_mp_body,
    mesh=_mesh,
    compiler_params=_sc_params,
    out_type=jax.ShapeDtypeStruct((2 * NPAD, D), jnp.float32),
    scratch_types=[
        pltpu.VMEM((SCH, CHUNK), jnp.int32),
        pltpu.VMEM((SCH, CHUNK), jnp.int32),
        pltpu.VMEM((CHUNK, D), jnp.float32),
        pltpu.VMEM((CHUNK, D), jnp.float32),
        pltpu.VMEM((16, D), jnp.float32),
        pltpu.VMEM_SHARED((NPAD, D), jnp.float32),
        pltpu.SemaphoreType.DMA,
        pltpu.SemaphoreType.DMA,
    ],
)


def _sc_edge_body(ab, srcp, dstp, ew_out, degw_out, abv, sbuf, dbuf, ewf,
                  zbuf, acc):
    c = lax.axis_index("c")
    s = lax.axis_index("s")
    w = c * NSUB + s
    for k in range(RPS // 16):
        zbuf[pl.ds(k * 16, 16)] = _zero16()
    pltpu.sync_copy(zbuf, acc.at[pl.ds(s * RPS, RPS)])
    plsc.subcore_barrier()
    pltpu.sync_copy(ab, abv)
    pltpu.sync_copy(srcp.at[pl.ds(w * NCH, NCH)], sbuf)
    pltpu.sync_copy(dstp.at[pl.ds(w * NCH, NCH)], dbuf)
    c1 = jnp.full((16,), 1, jnp.int32)
    c2 = jnp.full((16,), 2, jnp.int32)
    c3 = jnp.full((16,), 3, jnp.int32)

    def lp(j, cr):
        for k in range(CHUNK // 16):
            sv4 = sbuf[j, pl.ds(k * 16, 16)] * 4
            dv4 = dbuf[j, pl.ds(k * 16, 16)] * 4
            ga0 = plsc.load_gather(abv, [sv4])
            ga1 = plsc.load_gather(abv, [sv4 + c1])
            gb0 = plsc.load_gather(abv, [dv4 + c2])
            gb1 = plsc.load_gather(abv, [dv4 + c3])
            w0 = 1.0 / (1.0 + jnp.exp(-(ga0 + gb0)))
            w1 = 1.0 / (1.0 + jnp.exp(-(ga1 + gb1)))
            ewf[pl.ds(j * CHUNK + k * 16, 16)] = 0.5 * (w0 + w1)
        pltpu.sync_copy(ewf.at[pl.ds(j * CHUNK, CHUNK)],
                        acc.at[dbuf.at[j]], add=True)
        return cr

    lax.fori_loop(0, NCH, lp, 0)
    pltpu.sync_copy(ewf, ew_out.at[pl.ds(w * EPW, EPW)])
    plsc.subcore_barrier()
    pltpu.sync_copy(acc.at[pl.ds(s * RPS, RPS)], zbuf)
    pltpu.sync_copy(zbuf, degw_out.at[pl.ds(c * NPAD + s * RPS, RPS)])


_sc_edge = pl.kernel(
    _sc_edge_body,
    mesh=_mesh,
    compiler_params=_sc_params,
    out_type=[
        jax.ShapeDtypeStruct((EPAD,), jnp.float32),
        jax.ShapeDtypeStruct((2 * NPAD,), jnp.float32),
    ],
    scratch_types=[
        pltpu.VMEM((NPAD * 4,), jnp.float32),
        pltpu.VMEM((NCH, CHUNK), jnp.int32),
        pltpu.VMEM((NCH, CHUNK), jnp.int32),
        pltpu.VMEM((EPW,), jnp.float32),
        pltpu.VMEM((RPS,), jnp.float32),
        pltpu.VMEM_SHARED((NPAD,), jnp.float32),
    ],
)


def _sc_map_body(hm5, srcp, dstp, ewp, out, hmv, sbuf, dbuf, ewf, cb, zbuf,
                 acc0, acc1, acc2, acc3, acc4):
    accs = (acc0, acc1, acc2, acc3, acc4)
    c = lax.axis_index("c")
    s = lax.axis_index("s")
    w = c * NSUB + s
    for k in range(RPS // 16):
        zbuf[pl.ds(k * 16, 16)] = _zero16()
    for a in accs:
        pltpu.sync_copy(zbuf, a.at[pl.ds(s * RPS, RPS)])
    plsc.subcore_barrier()
    pltpu.sync_copy(hm5, hmv)
    pltpu.sync_copy(srcp.at[pl.ds(w * NCH, NCH)], sbuf)
    pltpu.sync_copy(dstp.at[pl.ds(w * NCH, NCH)], dbuf)
    pltpu.sync_copy(ewp.at[pl.ds(w * EPW, EPW)], ewf)
    cc = [jnp.full((16,), i, jnp.int32) for i in range(NOPS)]

    def lp(j, cr):
        for k in range(CHUNK // 16):
            sv5 = sbuf[j, pl.ds(k * 16, 16)] * NOPS
            ev = ewf[pl.ds(j * CHUNK + k * 16, 16)]
            for ch in range(NOPS):
                g = plsc.load_gather(hmv, [sv5 + cc[ch]])
                cb[ch, pl.ds(k * 16, 16)] = g * ev
        for ch in range(NOPS):
            pltpu.sync_copy(cb.at[ch], accs[ch].at[dbuf.at[j]], add=True)
        return cr

    lax.fori_loop(0, NCH, lp, 0)
    plsc.subcore_barrier()
    for ch in range(NOPS):
        pltpu.sync_copy(accs[ch].at[pl.ds(s * RPS, RPS)], zbuf)
        pltpu.sync_copy(
            zbuf, out.at[pl.ds((c * NOPS + ch) * NPAD + s * RPS, RPS)])


_sc_map = pl.kernel(
    _sc_map_body,
    mesh=_mesh,
    compiler_params=_sc_params,
    out_type=jax.ShapeDtypeStruct((2 * NOPS * NPAD,), jnp.float32),
    scratch_types=[
        pltpu.VMEM((NPAD * NOPS,), jnp.float32),
        pltpu.VMEM((NCH, CHUNK), jnp.int32),
        pltpu.VMEM((NCH, CHUNK), jnp.int32),
        pltpu.VMEM((EPW,), jnp.float32),
        pltpu.VMEM((NOPS, CHUNK), jnp.float32),
        pltpu.VMEM((RPS,), jnp.float32),
        pltpu.VMEM_SHARED((NPAD,), jnp.float32),
        pltpu.VMEM_SHARED((NPAD,), jnp.float32),
        pltpu.VMEM_SHARED((NPAD,), jnp.float32),
        pltpu.VMEM_SHARED((NPAD,), jnp.float32),
        pltpu.VMEM_SHARED((NPAD,), jnp.float32),
    ],
)


def _tc1_body(x_ref, w_ref, degt_ref, hw_ref, dinv_ref):
    dt = degt_ref[...]
    dinv = lax.rsqrt(dt[:, 0:1] + dt[:, 1:2] + 1.0)
    t = jnp.dot(x_ref[...], w_ref[...], preferred_element_type=jnp.float32)
    hw_ref[...] = t * dinv
    dinv_ref[...] = dinv


_tc1 = pl.pallas_call(
    _tc1_body,
    grid=(GRID,),
    in_specs=[
        pl.BlockSpec((BR, D), lambda i: (i, 0)),
        pl.BlockSpec((D, D), lambda i: (0, 0)),
        pl.BlockSpec((BR, 2), lambda i: (i, 0)),
    ],
    out_specs=[
        pl.BlockSpec((BR, D), lambda i: (i, 0)),
        pl.BlockSpec((BR, 1), lambda i: (i, 0)),
    ],
    out_shape=[
        jax.ShapeDtypeStruct((NPAD, D), jnp.float32),
        jax.ShapeDtypeStruct((NPAD, 1), jnp.float32),
    ],
)


def _tc2_body(p0_ref, p1_ref, hw_ref, dinv_ref, b_ref, w2_ref, out_ref):
    dinv = dinv_ref[...]
    h = jnp.maximum(
        dinv * (p0_ref[...] + p1_ref[...] + hw_ref[...]) + b_ref[...], 0.0)
    out_ref[...] = jnp.dot(
        h, w2_ref[...], preferred_element_type=jnp.float32) * dinv


_tc2 = pl.pallas_call(
    _tc2_body,
    grid=(GRID,),
    in_specs=[
        pl.BlockSpec((BR, D), lambda i: (i, 0)),
        pl.BlockSpec((BR, D), lambda i: (i, 0)),
        pl.BlockSpec((BR, D), lambda i: (i, 0)),
        pl.BlockSpec((BR, 1), lambda i: (i, 0)),
        pl.BlockSpec((1, D), lambda i: (0, 0)),
        pl.BlockSpec((D, D), lambda i: (0, 0)),
    ],
    out_specs=pl.BlockSpec((BR, D), lambda i: (i, 0)),
    out_shape=jax.ShapeDtypeStruct((NPAD, D), jnp.float32),
)


def _tc3_body(q0_ref, q1_ref, hw_ref, dinv_ref, b_ref, wet_ref, bet_ref,
              wmp_ref, ab_ref, hmr_ref):
    dinv = dinv_ref[...]
    h = jnp.maximum(
        dinv * (q0_ref[...] + q1_ref[...] + hw_ref[...]) + b_ref[...], 0.0)
    ab_ref[...] = jnp.dot(
        h, wet_ref[...], preferred_element_type=jnp.float32) + bet_ref[...]
    hmr_ref[...] = jnp.dot(h, wmp_ref[...], preferred_element_type=jnp.float32)


_tc3 = pl.pallas_call(
    _tc3_body,
    grid=(GRID,),
    in_specs=[
        pl.BlockSpec((BR, D), lambda i: (i, 0)),
        pl.BlockSpec((BR, D), lambda i: (i, 0)),
        pl.BlockSpec((BR, D), lambda i: (i, 0)),
        pl.BlockSpec((BR, 1), lambda i: (i, 0)),
        pl.BlockSpec((1, D), lambda i: (0, 0)),
        pl.BlockSpec((D, 4), lambda i: (0, 0)),
        pl.BlockSpec((1, 4), lambda i: (0, 0)),
        pl.BlockSpec((D, NOPS), lambda i: (0, 0)),
    ],
    out_specs=[
        pl.BlockSpec((BR, 4), lambda i: (i, 0)),
        pl.BlockSpec((BR, NOPS), lambda i: (i, 0)),
    ],
    out_shape=[
        jax.ShapeDtypeStruct((NPAD, 4), jnp.float32),
        jax.ShapeDtypeStruct((NPAD, NOPS), jnp.float32),
    ],
)


def _tc4_body(degwt_ref, hmr_ref, dinvw_ref, hmp_ref):
    dt = degwt_ref[...]
    dv = lax.rsqrt(dt[:, 0:1] + dt[:, 1:2] + 1.0)
    dinvw_ref[...] = dv
    hmp_ref[...] = hmr_ref[...] * dv


_tc4 = pl.pallas_call(
    _tc4_body,
    grid=(GRID,),
    in_specs=[
        pl.BlockSpec((BR, 2), lambda i: (i, 0)),
        pl.BlockSpec((BR, NOPS), lambda i: (i, 0)),
    ],
    out_specs=[
        pl.BlockSpec((BR, 1), lambda i: (i, 0)),
        pl.BlockSpec((BR, NOPS), lambda i: (i, 0)),
    ],
    out_shape=[
        jax.ShapeDtypeStruct((NPAD, 1), jnp.float32),
        jax.ShapeDtypeStruct((NPAD, NOPS), jnp.float32),
    ],
)


def _tc5_body(r0_ref, r1_ref, hmp_ref, dinvw_ref, bm_ref, g_ref, out_ref):
    op = dinvw_ref[...] * (r0_ref[...] + r1_ref[...] + hmp_ref[...]) + bm_ref[...]
    z = (op + g_ref[...]) / 1.0
    m = jnp.max(z, axis=1, keepdims=True)
    e = jnp.exp(z - m)
    ys = e / jnp.sum(e, axis=1, keepdims=True)
    oh = (z == m).astype(jnp.float32)
    out_ref[...] = ys + (oh - ys)


_tc5 = pl.pallas_call(
    _tc5_body,
    grid=(GRID,),
    in_specs=[
        pl.BlockSpec((BR, NOPS), lambda i: (i, 0)),
        pl.BlockSpec((BR, NOPS), lambda i: (i, 0)),
        pl.BlockSpec((BR, NOPS), lambda i: (i, 0)),
        pl.BlockSpec((BR, 1), lambda i: (i, 0)),
        pl.BlockSpec((1, NOPS), lambda i: (0, 0)),
        pl.BlockSpec((BR, NOPS), lambda i: (i, 0)),
    ],
    out_specs=pl.BlockSpec((BR, NOPS), lambda i: (i, 0)),
    out_shape=jax.ShapeDtypeStruct((NPAD, NOPS), jnp.float32),
)


def kernel(x, edge_index, batch, W1, b1, W2, b2, We, be, Wm, bm):
    del batch
    f32 = jnp.float32
    src = edge_index[0].astype(jnp.int32)
    dst = edge_index[1].astype(jnp.int32)
    pad = jnp.full((EPAD - E,), N, jnp.int32)
    srcp = jnp.concatenate([src, pad]).reshape(EROWS, CHUNK)
    dstp = jnp.concatenate([dst, pad]).reshape(EROWS, CHUNK)
    xp = jnp.zeros((NPAD, D), f32).at[:N].set(x)

    b1r = b1.reshape(1, D)
    b2r = b2.reshape(1, D)
    wet = jnp.concatenate([We[:D], We[D:]], axis=1)
    bet = jnp.concatenate([be, jnp.zeros((2,), f32)]).reshape(1, 4)
    wmp = Wm
    bmp = bm.reshape(1, NOPS)
    u = jax.random.uniform(jax.random.key(42), (N, NOPS), dtype=f32,
                           minval=1e-6, maxval=1.0 - 1e-6)
    g = -jnp.log(-jnp.log(u))
    gp = jnp.zeros((NPAD, NOPS), f32).at[:N].set(g)

    degp = _sc_deg(dstp).reshape(2, NPAD)
    hw1, dinv = _tc1(xp, W1, jnp.transpose(degp))
    p = _sc_mp(hw1, srcp, dstp).reshape(2, NPAD, D)
    hw2 = _tc2(p[0], p[1], hw1, dinv, b1r, W2)
    q = _sc_mp(hw2, srcp, dstp).reshape(2, NPAD, D)
    ab, hmr = _tc3(q[0], q[1], hw2, dinv, b2r, wet, bet, wmp)
    ew, degwp = _sc_edge(ab.reshape(NPAD * 4), srcp, dstp)
    degwp = degwp.reshape(2, NPAD)
    dinvw, hmp = _tc4(jnp.transpose(degwp), hmr)
    r = _sc_map(hmp.reshape(NPAD * NOPS), srcp, dstp, ew).reshape(2, NOPS, NPAD)
    outp = _tc5(jnp.transpose(r[0]), jnp.transpose(r[1]), hmp, dinvw, bmp, gp)
    return outp[:N]
